# Initial kernel scaffold; baseline (speedup 1.0000x reference)
import jax
import jax.numpy as jnp
from jax.experimental import pallas as pl


def kernel(inputs):
    raise NotImplementedError("write your pallas kernel here")


# speedup vs baseline: 2.9270x; 2.9270x over previous
import math

import jax
import jax.numpy as jnp
import numpy as np
from jax.experimental import pallas as pl
from jax.experimental.pallas import tpu as pltpu

_D_MODEL = 768
_S = 32
_W = 256
_CTX = 3 * _W


def _build_pe():
    pe = np.zeros((_CTX, _D_MODEL), dtype=np.float32)
    position = np.arange(0, _CTX, dtype=np.float32)[:, None]
    div_term = np.exp(
        np.arange(0, _D_MODEL, 2, dtype=np.float32) * (-math.log(10000.0) / _D_MODEL)
    )
    pe[:, 0::2] = np.sin(position * div_term)
    pe[:, 1::2] = np.cos(position * div_term)
    return jnp.asarray(pe)


_PE = _build_pe()


def _top3_kernel(first_ref, idx_ref):
    f = first_ref[...]
    d = jnp.sum(jnp.abs(f[:, None, :] - f[None, :, :]), axis=-1)
    iota = jax.lax.broadcasted_iota(jnp.int32, (_S, _S), 1)
    inf = jnp.float32(jnp.inf)
    i0 = jnp.argmin(d, axis=1).astype(jnp.int32)
    d1 = jnp.where(iota == i0[:, None], inf, d)
    i1 = jnp.argmin(d1, axis=1).astype(jnp.int32)
    d2 = jnp.where(iota == i1[:, None], inf, d1)
    i2 = jnp.argmin(d2, axis=1).astype(jnp.int32)
    idx_ref[...] = jnp.stack([i0, i1, i2], axis=1)


def _attn_kernel(idx_ref, a_ref, b_ref, c_ref, pe_ref, o_ref):
    del idx_ref
    pe = pe_ref[...]
    a = a_ref[0] + pe[:_W]
    b = b_ref[0] + pe[_W : 2 * _W]
    c = c_ref[0] + pe[2 * _W :]
    q = a
    dn = (((1,), (1,)), ((), ()))
    s = jnp.concatenate(
        [
            jax.lax.dot_general(q, a, dn, preferred_element_type=jnp.float32),
            jax.lax.dot_general(q, b, dn, preferred_element_type=jnp.float32),
            jax.lax.dot_general(q, c, dn, preferred_element_type=jnp.float32),
        ],
        axis=1,
    ) * jnp.float32(1.0 / math.sqrt(_D_MODEL))
    m = jnp.max(s, axis=1, keepdims=True)
    e = jnp.exp(s - m)
    p = e / jnp.sum(e, axis=1, keepdims=True)
    o = (
        jnp.dot(p[:, :_W], a, preferred_element_type=jnp.float32)
        + jnp.dot(p[:, _W : 2 * _W], b, preferred_element_type=jnp.float32)
        + jnp.dot(p[:, 2 * _W :], c, preferred_element_type=jnp.float32)
    )
    o_ref[0] = o


def kernel(inputs):
    first = inputs[:, 0, :]
    top3 = pl.pallas_call(
        _top3_kernel,
        out_shape=jax.ShapeDtypeStruct((_S, 3), jnp.int32),
    )(first)

    grid_spec = pltpu.PrefetchScalarGridSpec(
        num_scalar_prefetch=1,
        grid=(_S,),
        in_specs=[
            pl.BlockSpec((1, _W, _D_MODEL), lambda i, idx: (idx[i, 0], 0, 0)),
            pl.BlockSpec((1, _W, _D_MODEL), lambda i, idx: (idx[i, 1], 0, 0)),
            pl.BlockSpec((1, _W, _D_MODEL), lambda i, idx: (idx[i, 2], 0, 0)),
            pl.BlockSpec((_CTX, _D_MODEL), lambda i, idx: (0, 0)),
        ],
        out_specs=pl.BlockSpec((1, _W, _D_MODEL), lambda i, idx: (i, 0, 0)),
    )
    out = pl.pallas_call(
        _attn_kernel,
        grid_spec=grid_spec,
        out_shape=jax.ShapeDtypeStruct((_S, _W, _D_MODEL), jnp.float32),
    )(top3, inputs, inputs, inputs, _PE)
    return out[:, : _W - 1, :]


# speedup vs baseline: 3.2521x; 1.1111x over previous
import math

import jax
import jax.numpy as jnp
import numpy as np
from jax.experimental import pallas as pl
from jax.experimental.pallas import tpu as pltpu

_D_MODEL = 768
_S = 32
_W = 256
_CTX = 3 * _W


def _build_pe():
    pe = np.zeros((_CTX, _D_MODEL), dtype=np.float32)
    position = np.arange(0, _CTX, dtype=np.float32)[:, None]
    div_term = np.exp(
        np.arange(0, _D_MODEL, 2, dtype=np.float32) * (-math.log(10000.0) / _D_MODEL)
    )
    pe[:, 0::2] = np.sin(position * div_term)
    pe[:, 1::2] = np.cos(position * div_term)
    return jnp.asarray(pe)


_PE = _build_pe()


def _top3_kernel(first_ref, idx_ref):
    f = first_ref[...]
    d = jnp.sum(jnp.abs(f[:, None, :] - f[None, :, :]), axis=-1)
    iota = jax.lax.broadcasted_iota(jnp.int32, (_S, _S), 1)
    inf = jnp.float32(jnp.inf)
    i0 = jnp.argmin(d, axis=1).astype(jnp.int32)
    d1 = jnp.where(iota == i0[:, None], inf, d)
    i1 = jnp.argmin(d1, axis=1).astype(jnp.int32)
    d2 = jnp.where(iota == i1[:, None], inf, d1)
    i2 = jnp.argmin(d2, axis=1).astype(jnp.int32)
    idx_ref[...] = jnp.stack([i0, i1, i2], axis=1)


def _attn_kernel(idx_ref, a_ref, b_ref, c_ref, pe_ref, o_ref):
    del idx_ref
    pe = pe_ref[...]
    a = a_ref[0] + pe[:_W]
    b = b_ref[0] + pe[_W : 2 * _W]
    c = c_ref[0] + pe[2 * _W :]
    q = a
    dn = (((1,), (1,)), ((), ()))
    s = jnp.concatenate(
        [
            jax.lax.dot_general(q, a, dn, preferred_element_type=jnp.float32),
            jax.lax.dot_general(q, b, dn, preferred_element_type=jnp.float32),
            jax.lax.dot_general(q, c, dn, preferred_element_type=jnp.float32),
        ],
        axis=1,
    ) * jnp.float32(1.0 / math.sqrt(_D_MODEL))
    m = jnp.max(s, axis=1, keepdims=True)
    e = jnp.exp(s - m)
    p = e / jnp.sum(e, axis=1, keepdims=True)
    o = (
        jnp.dot(p[:, :_W], a, preferred_element_type=jnp.float32)
        + jnp.dot(p[:, _W : 2 * _W], b, preferred_element_type=jnp.float32)
        + jnp.dot(p[:, 2 * _W :], c, preferred_element_type=jnp.float32)
    )
    o_ref[0] = o[: _W - 1]


def kernel(inputs):
    first = inputs[:, 0, :]
    top3 = pl.pallas_call(
        _top3_kernel,
        out_shape=jax.ShapeDtypeStruct((_S, 3), jnp.int32),
    )(first)

    grid_spec = pltpu.PrefetchScalarGridSpec(
        num_scalar_prefetch=1,
        grid=(_S,),
        in_specs=[
            pl.BlockSpec((1, _W, _D_MODEL), lambda i, idx: (idx[i, 0], 0, 0)),
            pl.BlockSpec((1, _W, _D_MODEL), lambda i, idx: (idx[i, 1], 0, 0)),
            pl.BlockSpec((1, _W, _D_MODEL), lambda i, idx: (idx[i, 2], 0, 0)),
            pl.BlockSpec((_CTX, _D_MODEL), lambda i, idx: (0, 0)),
        ],
        out_specs=pl.BlockSpec((1, _W - 1, _D_MODEL), lambda i, idx: (i, 0, 0)),
    )
    out = pl.pallas_call(
        _attn_kernel,
        grid_spec=grid_spec,
        out_shape=jax.ShapeDtypeStruct((_S, _W - 1, _D_MODEL), jnp.float32),
    )(top3, inputs, inputs, inputs, _PE)
    return out
